# baseline (speedup 1.0000x reference)
import functools

import jax
import jax.numpy as jnp
from jax.experimental import pallas as pl
from jax.experimental.pallas import tpu as pltpu

_N = 10000
_D = 128
_BI = 200
_NB = _N // _BI
_NCACHE = 8


def _is_cached(b):
    return (b % 2 == 0) & (b >= _NB - 2 * _NCACHE) & (b <= _NB - 2)


def _slot(b):
    return (_NB - 2 - b) // 2


def _gcn_kernel(x_ref, w1t_ref, b1_ref, w2t_ref, b2_ref, a_ref,
                o_ref, y_ref, g_ref, c_ref):
    i = pl.program_id(0)
    j = 2 * _NB - 1 - i

    @pl.when(i == 0)
    def _():
        y_ref[...] = jnp.dot(x_ref[...], w1t_ref[...],
                             preferred_element_type=jnp.float32)

    @pl.when(i < _NB)
    def _():
        h = jnp.dot(a_ref[...], y_ref[...],
                    preferred_element_type=jnp.float32)
        h = jnp.maximum(h + b1_ref[...], 0.0)
        g_ref[pl.ds(i * _BI, _BI), :] = jnp.dot(
            h, w2t_ref[...], preferred_element_type=jnp.float32)

    @pl.when((i < _NB) & _is_cached(i))
    def _():
        c_ref[pl.ds(_slot(i) * _BI, _BI), :] = a_ref[...].astype(jnp.bfloat16)

    @pl.when((i >= _NB) & jnp.logical_not(_is_cached(j)))
    def _():
        o_ref[...] = jnp.dot(a_ref[...], g_ref[...],
                             preferred_element_type=jnp.float32) + b2_ref[...]

    @pl.when((i >= _NB) & _is_cached(j))
    def _():
        a_c = c_ref[pl.ds(_slot(j) * _BI, _BI), :]
        o_ref[...] = jnp.dot(a_c, g_ref[...].astype(jnp.bfloat16),
                             preferred_element_type=jnp.float32) + b2_ref[...]


def _a_index_map(i):
    j = 2 * _NB - 1 - i
    j = jnp.where(_is_cached(j), j + 1, j)
    return (jnp.where(i < _NB, i, j), 0)


def _o_index_map(i):
    return (jnp.where(i < _NB, 0, 2 * _NB - 1 - i), 0)


@functools.partial(jax.jit, static_argnames=())
def kernel(x, adj, W1, b1, W2, b2):
    n, d = adj.shape[0], x.shape[1]
    nb = n // _BI
    b1r = b1.reshape(1, -1)
    b2r = b2.reshape(1, -1)

    out = pl.pallas_call(
        _gcn_kernel,
        grid=(2 * nb,),
        in_specs=[
            pl.BlockSpec((n, d), lambda i: (0, 0)),
            pl.BlockSpec((d, d), lambda i: (0, 0)),
            pl.BlockSpec((1, d), lambda i: (0, 0)),
            pl.BlockSpec((d, d), lambda i: (0, 0)),
            pl.BlockSpec((1, d), lambda i: (0, 0)),
            pl.BlockSpec((_BI, n), _a_index_map),
        ],
        out_specs=pl.BlockSpec((_BI, d), _o_index_map),
        out_shape=jax.ShapeDtypeStruct((n, d), jnp.float32),
        scratch_shapes=[
            pltpu.VMEM((n, d), jnp.float32),
            pltpu.VMEM((n, d), jnp.float32),
            pltpu.VMEM((_NCACHE * _BI, n), jnp.bfloat16),
        ],
        compiler_params=pltpu.CompilerParams(
            dimension_semantics=("arbitrary",),
            vmem_limit_bytes=64 * 1024 * 1024,
        ),
    )(x, W1.T, b1r, W2.T, b2r, adj)

    return out


# speedup vs baseline: 1.0005x; 1.0005x over previous
import functools

import jax
import jax.numpy as jnp
from jax.experimental import pallas as pl
from jax.experimental.pallas import tpu as pltpu

_N = 10000
_D = 128
_BI = 200
_NB = _N // _BI
_NCACHE = 7


def _is_cached(b):
    return (b % 2 == 0) & (b >= _NB - 2 * _NCACHE) & (b <= _NB - 2)


def _slot(b):
    return (_NB - 2 - b) // 2


def _gcn_kernel(x_ref, w1t_ref, b1_ref, w2t_ref, b2_ref, a_ref,
                o_ref, y_ref, g_ref, g16_ref, c_ref):
    i = pl.program_id(0)
    j = 2 * _NB - 1 - i

    @pl.when(i == 0)
    def _():
        y_ref[...] = jnp.dot(x_ref[...], w1t_ref[...],
                             preferred_element_type=jnp.float32)

    @pl.when(i < _NB)
    def _():
        h = jnp.dot(a_ref[...], y_ref[...],
                    preferred_element_type=jnp.float32)
        h = jnp.maximum(h + b1_ref[...], 0.0)
        g_ref[pl.ds(i * _BI, _BI), :] = jnp.dot(
            h, w2t_ref[...], preferred_element_type=jnp.float32)

    @pl.when((i < _NB) & _is_cached(i))
    def _():
        c_ref[_slot(i)] = a_ref[...].astype(jnp.bfloat16)

    @pl.when(i == _NB)
    def _():
        g16_ref[...] = g_ref[...].astype(jnp.bfloat16)

    @pl.when((i >= _NB) & jnp.logical_not(_is_cached(j)))
    def _():
        o_ref[...] = jnp.dot(a_ref[...], g_ref[...],
                             preferred_element_type=jnp.float32) + b2_ref[...]

    @pl.when((i >= _NB) & _is_cached(j))
    def _():
        o_ref[...] = jnp.dot(c_ref[_slot(j)], g16_ref[...],
                             preferred_element_type=jnp.float32) + b2_ref[...]


def _a_index_map(i):
    j = 2 * _NB - 1 - i
    j = jnp.where(_is_cached(j), j + 1, j)
    return (jnp.where(i < _NB, i, j), 0)


def _o_index_map(i):
    return (jnp.where(i < _NB, 0, 2 * _NB - 1 - i), 0)


@functools.partial(jax.jit, static_argnames=())
def kernel(x, adj, W1, b1, W2, b2):
    n, d = adj.shape[0], x.shape[1]
    nb = n // _BI
    b1r = b1.reshape(1, -1)
    b2r = b2.reshape(1, -1)

    out = pl.pallas_call(
        _gcn_kernel,
        grid=(2 * nb,),
        in_specs=[
            pl.BlockSpec((n, d), lambda i: (0, 0)),
            pl.BlockSpec((d, d), lambda i: (0, 0)),
            pl.BlockSpec((1, d), lambda i: (0, 0)),
            pl.BlockSpec((d, d), lambda i: (0, 0)),
            pl.BlockSpec((1, d), lambda i: (0, 0)),
            pl.BlockSpec((_BI, n), _a_index_map),
        ],
        out_specs=pl.BlockSpec((_BI, d), _o_index_map),
        out_shape=jax.ShapeDtypeStruct((n, d), jnp.float32),
        scratch_shapes=[
            pltpu.VMEM((n, d), jnp.float32),
            pltpu.VMEM((n, d), jnp.float32),
            pltpu.VMEM((n, d), jnp.bfloat16),
            pltpu.VMEM((_NCACHE, _BI, n), jnp.bfloat16),
        ],
        compiler_params=pltpu.CompilerParams(
            dimension_semantics=("arbitrary",),
            vmem_limit_bytes=64 * 1024 * 1024,
        ),
    )(x, W1.T, b1r, W2.T, b2r, adj)

    return out
